# baseline (speedup 1.0000x reference)
import functools

import jax
import jax.numpy as jnp
from jax.experimental import pallas as pl
from jax.experimental.pallas import tpu as pltpu

_B = 16384
_C = 1000
_BL = 2048


def _fused_kernel(x_ref, yl_ref, yp_ref, part_ref):
    x = x_ref[...]
    yl = yl_ref[...]
    m = jnp.max(x, axis=0, keepdims=True)
    e = jnp.exp(x - m)
    s = jnp.sum(e, axis=0, keepdims=True)
    yp_ref[...] = jnp.clip(e * (1.0 / s), 1e-4, 1.0 - 1e-4)
    lse = m + jnp.log(s)
    ce = lse * jnp.sum(yl, axis=0, keepdims=True) \
        - jnp.sum(yl * x, axis=0, keepdims=True)
    part_ref[0, 0, 0] = jnp.sum(ce)


@functools.partial(jax.jit, static_argnums=())
def _run(output, y_labeled):
    xt = output.T
    ylt = y_labeled.T
    grid = (_B // _BL,)
    yp_t, partials = pl.pallas_call(
        _fused_kernel,
        grid=grid,
        in_specs=[
            pl.BlockSpec((_C, _BL), lambda i: (0, i)),
            pl.BlockSpec((_C, _BL), lambda i: (0, i)),
        ],
        out_specs=[
            pl.BlockSpec((_C, _BL), lambda i: (0, i)),
            pl.BlockSpec((1, 1, 1), lambda i: (i, 0, 0), memory_space=pltpu.SMEM),
        ],
        out_shape=[
            jax.ShapeDtypeStruct((_C, _B), jnp.float32),
            jax.ShapeDtypeStruct((grid[0], 1, 1), jnp.float32),
        ],
        compiler_params=pltpu.CompilerParams(
            dimension_semantics=("parallel",),
        ),
    )(xt, ylt)
    return jnp.sum(partials) * (1.0 / _B), yp_t.T


def kernel(iteration, output, y_labeled):
    del iteration
    final_loss, y_pred = _run(output, y_labeled)
    return (final_loss, y_pred)


# speedup vs baseline: 1.0456x; 1.0456x over previous
import functools

import jax
import jax.numpy as jnp
from jax.experimental import pallas as pl
from jax.experimental.pallas import tpu as pltpu

_B = 16384
_C = 1000
_BL = 2048


def _fused_kernel(x_ref, yl_ref, yp_ref, loss_ref):
    i = pl.program_id(0)
    x = x_ref[...]
    yl = yl_ref[...]
    m = jnp.max(x, axis=0, keepdims=True)
    e = jnp.exp(x - m)
    s = jnp.sum(e, axis=0, keepdims=True)
    yp_ref[...] = jnp.clip(e * (1.0 / s), 1e-4, 1.0 - 1e-4)
    lse = m + jnp.log(s)
    ce = lse * jnp.sum(yl, axis=0, keepdims=True) \
        - jnp.sum(yl * x, axis=0, keepdims=True)
    part = jnp.sum(ce) * (1.0 / _B)

    @pl.when(i == 0)
    def _init():
        loss_ref[0, 0] = part

    @pl.when(i > 0)
    def _acc():
        loss_ref[0, 0] += part


@functools.partial(jax.jit, static_argnums=())
def _run(output, y_labeled):
    xt = output.T
    ylt = y_labeled.T
    grid = (_B // _BL,)
    yp_t, loss = pl.pallas_call(
        _fused_kernel,
        grid=grid,
        in_specs=[
            pl.BlockSpec((_C, _BL), lambda i: (0, i)),
            pl.BlockSpec((_C, _BL), lambda i: (0, i)),
        ],
        out_specs=[
            pl.BlockSpec((_C, _BL), lambda i: (0, i)),
            pl.BlockSpec((1, 1), lambda i: (0, 0), memory_space=pltpu.SMEM),
        ],
        out_shape=[
            jax.ShapeDtypeStruct((_C, _B), jnp.float32),
            jax.ShapeDtypeStruct((1, 1), jnp.float32),
        ],
    )(xt, ylt)
    return loss[0, 0], yp_t.T


def kernel(iteration, output, y_labeled):
    del iteration
    final_loss, y_pred = _run(output, y_labeled)
    return (final_loss, y_pred)


# speedup vs baseline: 1.0885x; 1.0411x over previous
import functools

import jax
import jax.numpy as jnp
from jax.experimental import pallas as pl
from jax.experimental.pallas import tpu as pltpu

_B = 16384
_C = 1000
_BL = 512
_K = 6
_NCH = _B // _BL


def _fused_kernel(x_hbm, yl_hbm, yp_hbm, loss_ref,
                  x_buf, yl_buf, yp_buf, sem_x, sem_yl, sem_out):
    def in_copies(i, s):
        cx = pltpu.make_async_copy(
            x_hbm.at[:, pl.ds(i * _BL, _BL)], x_buf.at[s], sem_x.at[s])
        cy = pltpu.make_async_copy(
            yl_hbm.at[:, pl.ds(i * _BL, _BL)], yl_buf.at[s], sem_yl.at[s])
        return cx, cy

    def out_copy(i, s):
        return pltpu.make_async_copy(
            yp_buf.at[s], yp_hbm.at[:, pl.ds(i * _BL, _BL)], sem_out.at[s])

    for s in range(_K):
        cx, cy = in_copies(s, s)
        cx.start()
        cy.start()

    acc = jnp.float32(0.0)
    for i in range(_NCH):
        s = i % _K
        cx, cy = in_copies(i, s)
        cx.wait()
        cy.wait()
        x = x_buf[s, :, :]
        yl = yl_buf[s, :, :]
        m = jnp.max(x, axis=0, keepdims=True)
        e = jnp.exp(x - m)
        ssum = jnp.sum(e, axis=0, keepdims=True)
        if i >= _K:
            out_copy(i - _K, s).wait()
        yp_buf[s, :, :] = jnp.clip(e * (1.0 / ssum), 1e-4, 1.0 - 1e-4)
        out_copy(i, s).start()
        lse = m + jnp.log(ssum)
        ce = lse * jnp.sum(yl, axis=0, keepdims=True) \
            - jnp.sum(yl * x, axis=0, keepdims=True)
        acc = acc + jnp.sum(ce)
        if i + _K < _NCH:
            cx2, cy2 = in_copies(i + _K, s)
            cx2.start()
            cy2.start()

    for i in range(_NCH - _K, _NCH):
        out_copy(i, i % _K).wait()
    loss_ref[0, 0] = acc * (1.0 / _B)


@functools.partial(jax.jit, static_argnums=())
def _run(output, y_labeled):
    xt = output.T
    ylt = y_labeled.T
    yp_t, loss = pl.pallas_call(
        _fused_kernel,
        in_specs=[
            pl.BlockSpec(memory_space=pl.ANY),
            pl.BlockSpec(memory_space=pl.ANY),
        ],
        out_specs=[
            pl.BlockSpec(memory_space=pl.ANY),
            pl.BlockSpec(memory_space=pltpu.SMEM),
        ],
        out_shape=[
            jax.ShapeDtypeStruct((_C, _B), jnp.float32),
            jax.ShapeDtypeStruct((1, 1), jnp.float32),
        ],
        scratch_shapes=[
            pltpu.VMEM((_K, _C, _BL), jnp.float32),
            pltpu.VMEM((_K, _C, _BL), jnp.float32),
            pltpu.VMEM((_K, _C, _BL), jnp.float32),
            pltpu.SemaphoreType.DMA((_K,)),
            pltpu.SemaphoreType.DMA((_K,)),
            pltpu.SemaphoreType.DMA((_K,)),
        ],
    )(xt, ylt)
    return loss[0, 0], yp_t.T


def kernel(iteration, output, y_labeled):
    del iteration
    final_loss, y_pred = _run(output, y_labeled)
    return (final_loss, y_pred)
